# baseline (speedup 1.0000x reference)
import functools

import jax
import jax.numpy as jnp
from jax import lax
from jax.experimental import pallas as pl
from jax.experimental.pallas import tpu as pltpu
from jax.experimental.pallas import tpu_sc as plsc

B = 2
N = 10000
E = 320000
C = 128

NC = 2
NS = 16
LANES = 16

E_PER_TILE = E // NS
CHUNK = 80
NCHUNK = E_PER_TILE // CHUNK
PAIRS = NCHUNK // 2

HCHUNK = 2000
NH_CHUNKS = E // HCHUNK

RBLK = 40
NBLK_TOT = N // RBLK
BLK_ROUNDS = (NBLK_TOT + NS - 1) // NS
LH_DATA = BLK_ROUNDS * RBLK
LH = LH_DATA + LANES


def _matmul_body(x_ref, th_ref, o_ref):
    o_ref[0] = jnp.dot(x_ref[0], th_ref[...], preferred_element_type=jnp.float32)


def _project(x, theta):
    RB = 1000
    return pl.pallas_call(
        _matmul_body,
        grid=(B, N // RB),
        in_specs=[
            pl.BlockSpec((1, RB, C), lambda b, i: (b, i, 0)),
            pl.BlockSpec((C, C), lambda b, i: (0, 0)),
        ],
        out_specs=pl.BlockSpec((1, RB, C), lambda b, i: (b, i, 0)),
        out_shape=jax.ShapeDtypeStruct((B, N, C), jnp.float32),
    )(x, theta)


def _sc_body(xt_hbm, nidx_hbm, eidx_hbm, bias_hbm, out_hbm, xe_hbm,
             acc_sh, srcr_v, dstr_v, rows0_v, rows1_v, hist_v, hidx_v,
             nrm_v, bias_v, gsem0, gsem1, ssem0, ssem1):
    c = lax.axis_index("c")
    s = lax.axis_index("s")
    ebase = s * E_PER_TILE
    row0 = c * N

    pltpu.sync_copy(bias_hbm, bias_v)
    ones16 = jnp.ones((LANES,), jnp.float32)
    lane16 = lax.iota(jnp.int32, LANES)

    def zero_acc():
        def zrow(r, _):
            for j in range(C // LANES):
                nrm_v[r, j * LANES:(j + 1) * LANES] = jnp.zeros((LANES,), jnp.float32)
            return 0
        lax.fori_loop(0, RBLK, zrow, 0)
        for i in range(BLK_ROUNDS):
            blk = i * NS + s

            @pl.when(blk < NBLK_TOT)
            def _():
                pltpu.sync_copy(nrm_v, acc_sh.at[pl.ds(blk * RBLK, RBLK)])

    def build_hist(dst_hbm):
        def zh(i, _):
            hist_v[pl.ds(i * LANES, LANES)] = jnp.zeros((LANES,), jnp.float32)
            return 0
        lax.fori_loop(0, LH // LANES, zh, 0)

        def hchunk(h, _):
            pltpu.sync_copy(dst_hbm.at[pl.ds(c * E + h * HCHUNK, HCHUNK)], hidx_v)

            def add_j(j, _):
                v = hidx_v[pl.ds(j * LANES, LANES)]
                blk = v // RBLK
                own = (blk % NS) == s
                loc = (blk // NS) * RBLK + (v % RBLK)
                tgt = jnp.where(own, loc, LH_DATA + lane16)
                plsc.addupdate_scatter(hist_v, [tgt], ones16)
                return 0
            lax.fori_loop(0, HCHUNK // LANES, add_j, 0)
            return 0
        lax.fori_loop(0, NH_CHUNKS, hchunk, 0)

    def stream_pass(tbl_hbm, src_hbm, dst_hbm):
        base = c * E + ebase
        idx_end = B * E - CHUNK

        def load_idx(k, ring_row):
            off = jnp.minimum(base + k * CHUNK, idx_end)
            pltpu.sync_copy(src_hbm.at[pl.ds(off, CHUNK)], srcr_v.at[ring_row])
            pltpu.sync_copy(dst_hbm.at[pl.ds(off, CHUNK)], dstr_v.at[ring_row])

            def bias_j(j, _):
                sl = pl.ds(j * LANES, LANES)
                srcr_v[ring_row, sl] = srcr_v[ring_row, sl] + row0
                return 0
            lax.fori_loop(0, CHUNK // LANES, bias_j, 0)

        def gather(k, rows_ref, gsem):
            return pltpu.async_copy(tbl_hbm.at[srcr_v.at[k % 4]], rows_ref, gsem)

        load_idx(0, 0)
        load_idx(1, 1)
        gather(0, rows0_v, gsem0)
        gather(1, rows1_v, gsem1)

        def pair_body(t, _):
            for sub, (rows_ref, gsem, ssem) in enumerate(
                    ((rows0_v, gsem0, ssem0), (rows1_v, gsem1, ssem1))):
                k = 2 * t + sub
                pltpu.make_async_copy(tbl_hbm.at[srcr_v.at[k % 4]], rows_ref, gsem).wait()
                sc = pltpu.async_copy(rows_ref, acc_sh.at[dstr_v.at[k % 4]], ssem, add=True)
                load_idx(k + 2, (k + 2) % 4)
                sc.wait()
                gather(k + 2, rows_ref, gsem)
            return 0
        lax.fori_loop(0, PAIRS, pair_body, 0)
        pltpu.make_async_copy(tbl_hbm.at[srcr_v.at[NCHUNK % 4]], rows0_v, gsem0).wait()
        pltpu.make_async_copy(tbl_hbm.at[srcr_v.at[(NCHUNK + 1) % 4]], rows1_v, gsem1).wait()

    def normalize(dst_hbm, add_bias):
        for i in range(BLK_ROUNDS):
            blk = i * NS + s

            @pl.when(blk < NBLK_TOT)
            def _():
                base = blk * RBLK
                pltpu.sync_copy(acc_sh.at[pl.ds(base, RBLK)], nrm_v)

                def nrow(r, _):
                    gi = jnp.full((LANES,), i * RBLK, jnp.int32) + r
                    d = plsc.load_gather(hist_v, [gi])
                    recip = jnp.where(d > 0.0, 1.0 / d, 0.0)
                    for j in range(C // LANES):
                        sl = pl.ds(j * LANES, LANES)
                        v = nrm_v[r, sl] * recip
                        if add_bias:
                            v = v + bias_v[sl]
                        nrm_v[r, sl] = v
                    return 0
                lax.fori_loop(0, RBLK, nrow, 0)
                pltpu.sync_copy(nrm_v, dst_hbm.at[pl.ds(row0 + base, RBLK)])

    zero_acc()
    plsc.subcore_barrier()
    build_hist(eidx_hbm)
    stream_pass(xt_hbm, nidx_hbm, eidx_hbm)
    plsc.subcore_barrier()
    normalize(xe_hbm, add_bias=False)
    zero_acc()
    plsc.subcore_barrier()
    build_hist(nidx_hbm)
    stream_pass(xe_hbm, eidx_hbm, nidx_hbm)
    plsc.subcore_barrier()
    normalize(out_hbm, add_bias=True)


def _build_sc_kernel(interpret=False):
    mesh = plsc.VectorSubcoreMesh(
        core_axis_name="c", subcore_axis_name="s", num_cores=NC, num_subcores=NS
    )
    return pl.kernel(
        _sc_body,
        out_type=(
            jax.ShapeDtypeStruct((B * N, C), jnp.float32),
            jax.ShapeDtypeStruct((B * N, C), jnp.float32),
        ),
        mesh=mesh,
        compiler_params=pltpu.CompilerParams(needs_layout_passes=False),
        scratch_types=[
            pltpu.VMEM_SHARED((N, C), jnp.float32),
            pltpu.VMEM((4, CHUNK), jnp.int32),
            pltpu.VMEM((4, CHUNK), jnp.int32),
            pltpu.VMEM((CHUNK, C), jnp.float32),
            pltpu.VMEM((CHUNK, C), jnp.float32),
            pltpu.VMEM((LH,), jnp.float32),
            pltpu.VMEM((HCHUNK,), jnp.int32),
            pltpu.VMEM((RBLK, C), jnp.float32),
            pltpu.VMEM((C,), jnp.float32),
            pltpu.SemaphoreType.DMA,
            pltpu.SemaphoreType.DMA,
            pltpu.SemaphoreType.DMA,
            pltpu.SemaphoreType.DMA,
        ],
        interpret=interpret,
    )


_hyconv_sc = _build_sc_kernel()


def kernel(x, H, theta, bias):
    xt = _project(x, theta).reshape(B * N, C)
    nidx = H[:, 0, :].reshape(-1)
    eidx = H[:, 1, :].reshape(-1)
    out, _ = _hyconv_sc(xt, nidx, eidx, bias)
    return out.reshape(B, N, C)


# speedup vs baseline: 2.7336x; 2.7336x over previous
import functools

import jax
import jax.numpy as jnp
from jax import lax
from jax.experimental import pallas as pl
from jax.experimental.pallas import tpu as pltpu
from jax.experimental.pallas import tpu_sc as plsc

B = 2
N = 10000
E = 320000
C = 128

NC = 2
NS = 16
LANES = 16

E_PER_TILE = E // NS
CHUNK = 80
NCHUNK = E_PER_TILE // CHUNK
PAIRS = NCHUNK // 2

HCHUNK = 8000
NH_CHUNKS = E // HCHUNK

ZBLK = 80
NZBLK = N // ZBLK
Z_ROUNDS = (NZBLK + NS - 1) // NS

RBLK = 16
NBLK_TOT = N // RBLK
BLK_ROUNDS = (NBLK_TOT + NS - 1) // NS
LH_DATA = BLK_ROUNDS * RBLK
LH = LH_DATA + LANES


def _matmul_body(x_ref, th_ref, o_ref):
    o_ref[0] = jnp.dot(x_ref[0], th_ref[...], preferred_element_type=jnp.float32)


def _project(x, theta):
    RB = 1000
    return pl.pallas_call(
        _matmul_body,
        grid=(B, N // RB),
        in_specs=[
            pl.BlockSpec((1, RB, C), lambda b, i: (b, i, 0)),
            pl.BlockSpec((C, C), lambda b, i: (0, 0)),
        ],
        out_specs=pl.BlockSpec((1, RB, C), lambda b, i: (b, i, 0)),
        out_shape=jax.ShapeDtypeStruct((B, N, C), jnp.float32),
    )(x, theta)


def _sc_body(xt_hbm, nidx_hbm, eidx_hbm, bias_hbm, out_hbm, xe_hbm,
             acc_sh, srcr_v, dstr_v, rows0_v, rows1_v, hist_v, hidx_v,
             nrm_v, bias_v, gsem0, gsem1, ssem0, ssem1):
    c = lax.axis_index("c")
    s = lax.axis_index("s")
    ebase = s * E_PER_TILE
    row0 = c * N

    pltpu.sync_copy(bias_hbm, bias_v)
    ones16 = jnp.ones((LANES,), jnp.float32)
    lane16 = lax.iota(jnp.int32, LANES)

    def zero_acc():
        def zrow(r, _):
            for j in range(C // LANES):
                rows0_v[r, j * LANES:(j + 1) * LANES] = jnp.zeros((LANES,), jnp.float32)
            return 0
        lax.fori_loop(0, ZBLK, zrow, 0)
        for i in range(Z_ROUNDS):
            blk = i * NS + s

            @pl.when(blk < NZBLK)
            def _():
                pltpu.sync_copy(rows0_v, acc_sh.at[pl.ds(blk * ZBLK, ZBLK)])

    def build_hist(dst_hbm):
        def zh(i, _):
            hist_v[pl.ds(i * LANES, LANES)] = jnp.zeros((LANES,), jnp.float32)
            return 0
        lax.fori_loop(0, LH // LANES, zh, 0)

        def hchunk(h, _):
            pltpu.sync_copy(dst_hbm.at[pl.ds(c * E + h * HCHUNK, HCHUNK)], hidx_v)

            def add_j(j, _):
                v = hidx_v[pl.ds(j * LANES, LANES)]
                own = (lax.shift_right_logical(v, 4) & (NS - 1)) == s
                loc = lax.shift_left(lax.shift_right_logical(v, 8), 4) + (v & (RBLK - 1))
                tgt = jnp.where(own, loc, LH_DATA + lane16)
                plsc.addupdate_scatter(hist_v, [tgt], ones16)
                return 0
            lax.fori_loop(0, HCHUNK // LANES, add_j, 0)
            return 0
        lax.fori_loop(0, NH_CHUNKS, hchunk, 0)

    def stream_pass(tbl_hbm, src_hbm, dst_hbm):
        base = c * E + ebase
        idx_end = B * E - CHUNK

        def load_idx(k, ring_row):
            off = jnp.minimum(base + k * CHUNK, idx_end)
            pltpu.sync_copy(src_hbm.at[pl.ds(off, CHUNK)], srcr_v.at[ring_row])
            pltpu.sync_copy(dst_hbm.at[pl.ds(off, CHUNK)], dstr_v.at[ring_row])

            def bias_j(j, _):
                sl = pl.ds(j * LANES, LANES)
                srcr_v[ring_row, sl] = srcr_v[ring_row, sl] + row0
                return 0
            lax.fori_loop(0, CHUNK // LANES, bias_j, 0)

        def gather(k, rows_ref, gsem):
            return pltpu.async_copy(tbl_hbm.at[srcr_v.at[k % 4]], rows_ref, gsem)

        load_idx(0, 0)
        load_idx(1, 1)
        gather(0, rows0_v, gsem0)
        gather(1, rows1_v, gsem1)

        def pair_body(t, _):
            for sub, (rows_ref, gsem, ssem) in enumerate(
                    ((rows0_v, gsem0, ssem0), (rows1_v, gsem1, ssem1))):
                k = 2 * t + sub
                pltpu.make_async_copy(tbl_hbm.at[srcr_v.at[k % 4]], rows_ref, gsem).wait()
                sc = pltpu.async_copy(rows_ref, acc_sh.at[dstr_v.at[k % 4]], ssem, add=True)
                load_idx(k + 2, (k + 2) % 4)
                sc.wait()
                gather(k + 2, rows_ref, gsem)
            return 0
        lax.fori_loop(0, PAIRS, pair_body, 0)
        pltpu.make_async_copy(tbl_hbm.at[srcr_v.at[NCHUNK % 4]], rows0_v, gsem0).wait()
        pltpu.make_async_copy(tbl_hbm.at[srcr_v.at[(NCHUNK + 1) % 4]], rows1_v, gsem1).wait()

    def normalize(dst_hbm, add_bias):
        def nround(i, _):
            blk = i * NS + s

            @pl.when(blk < NBLK_TOT)
            def _():
                base = blk * RBLK
                pltpu.sync_copy(acc_sh.at[pl.ds(base, RBLK)], nrm_v)

                def nrow(r, _):
                    gi = jnp.full((LANES,), 0, jnp.int32) + (i * RBLK + r)
                    d = plsc.load_gather(hist_v, [gi])
                    recip = jnp.where(d > 0.0, 1.0 / d, 0.0)
                    for j in range(C // LANES):
                        sl = pl.ds(j * LANES, LANES)
                        v = nrm_v[r, sl] * recip
                        if add_bias:
                            v = v + bias_v[sl]
                        nrm_v[r, sl] = v
                    return 0
                lax.fori_loop(0, RBLK, nrow, 0)
                pltpu.sync_copy(nrm_v, dst_hbm.at[pl.ds(row0 + base, RBLK)])
            return 0
        lax.fori_loop(0, BLK_ROUNDS, nround, 0)

    zero_acc()
    plsc.subcore_barrier()
    build_hist(eidx_hbm)
    stream_pass(xt_hbm, nidx_hbm, eidx_hbm)
    plsc.subcore_barrier()
    normalize(xe_hbm, add_bias=False)
    zero_acc()
    plsc.subcore_barrier()
    build_hist(nidx_hbm)
    stream_pass(xe_hbm, eidx_hbm, nidx_hbm)
    plsc.subcore_barrier()
    normalize(out_hbm, add_bias=True)


def _build_sc_kernel(interpret=False):
    mesh = plsc.VectorSubcoreMesh(
        core_axis_name="c", subcore_axis_name="s", num_cores=NC, num_subcores=NS
    )
    return pl.kernel(
        _sc_body,
        out_type=(
            jax.ShapeDtypeStruct((B * N, C), jnp.float32),
            jax.ShapeDtypeStruct((B * N, C), jnp.float32),
        ),
        mesh=mesh,
        compiler_params=pltpu.CompilerParams(needs_layout_passes=False),
        scratch_types=[
            pltpu.VMEM_SHARED((N, C), jnp.float32),
            pltpu.VMEM((4, CHUNK), jnp.int32),
            pltpu.VMEM((4, CHUNK), jnp.int32),
            pltpu.VMEM((CHUNK, C), jnp.float32),
            pltpu.VMEM((CHUNK, C), jnp.float32),
            pltpu.VMEM((LH,), jnp.float32),
            pltpu.VMEM((HCHUNK,), jnp.int32),
            pltpu.VMEM((RBLK, C), jnp.float32),
            pltpu.VMEM((C,), jnp.float32),
            pltpu.SemaphoreType.DMA,
            pltpu.SemaphoreType.DMA,
            pltpu.SemaphoreType.DMA,
            pltpu.SemaphoreType.DMA,
        ],
        interpret=interpret,
    )


_hyconv_sc = _build_sc_kernel()


def kernel(x, H, theta, bias):
    xt = _project(x, theta).reshape(B * N, C)
    nidx = H[:, 0, :].reshape(-1)
    eidx = H[:, 1, :].reshape(-1)
    out, _ = _hyconv_sc(xt, nidx, eidx, bias)
    return out.reshape(B, N, C)


# speedup vs baseline: 2.7596x; 1.0095x over previous
import functools

import jax
import jax.numpy as jnp
from jax import lax
from jax.experimental import pallas as pl
from jax.experimental.pallas import tpu as pltpu
from jax.experimental.pallas import tpu_sc as plsc

B = 2
N = 10000
E = 320000
C = 128

NC = 2
NS = 16
LANES = 16

E_PER_TILE = E // NS
CHUNK = 80
NCHUNK = E_PER_TILE // CHUNK
PAIRS = NCHUNK // 2

HCHUNK = 8000
NH_CHUNKS = E // HCHUNK

ZBLK = 80
NZBLK = N // ZBLK
Z_ROUNDS = (NZBLK + NS - 1) // NS

RBLK = 16
NBLK_TOT = N // RBLK
BLK_ROUNDS = (NBLK_TOT + NS - 1) // NS
LH_DATA = BLK_ROUNDS * RBLK
LH = LH_DATA + LANES


def _matmul_body(x_ref, th_ref, o_ref):
    o_ref[0] = jnp.dot(x_ref[0], th_ref[...], preferred_element_type=jnp.float32)


def _project(x, theta):
    RB = 1000
    return pl.pallas_call(
        _matmul_body,
        grid=(B, N // RB),
        in_specs=[
            pl.BlockSpec((1, RB, C), lambda b, i: (b, i, 0)),
            pl.BlockSpec((C, C), lambda b, i: (0, 0)),
        ],
        out_specs=pl.BlockSpec((1, RB, C), lambda b, i: (b, i, 0)),
        out_shape=jax.ShapeDtypeStruct((B, N, C), jnp.float32),
    )(x, theta)


def _sc_body(xt_hbm, nidx_hbm, eidx_hbm, bias_hbm, out_hbm, xe_hbm,
             acc_sh, srcr_v, dstr_v, rows0_v, rows1_v, hist_v, hidx_v,
             nrm_v, bias_v, gsem0, gsem1, ssem0, ssem1):
    c = lax.axis_index("c")
    s = lax.axis_index("s")
    ebase = s * E_PER_TILE
    row0 = c * N

    pltpu.sync_copy(bias_hbm, bias_v)
    ones16 = jnp.ones((LANES,), jnp.float32)
    lane16 = lax.iota(jnp.int32, LANES)

    def zero_acc():
        def zrow(r, _):
            for j in range(C // LANES):
                rows0_v[r, j * LANES:(j + 1) * LANES] = jnp.zeros((LANES,), jnp.float32)
            return 0
        lax.fori_loop(0, ZBLK, zrow, 0)
        for i in range(Z_ROUNDS):
            blk = i * NS + s

            @pl.when(blk < NZBLK)
            def _():
                pltpu.sync_copy(rows0_v, acc_sh.at[pl.ds(blk * ZBLK, ZBLK)])

    def zero_hist():
        def zh(i, _):
            hist_v[pl.ds(i * LANES, LANES)] = jnp.zeros((LANES,), jnp.float32)
            return 0
        lax.fori_loop(0, LH // LANES, zh, 0)

    def hist_chunk(h, dst_hbm):
        pltpu.sync_copy(dst_hbm.at[pl.ds(c * E + h * HCHUNK, HCHUNK)], hidx_v)

        def add_j(j, _):
            v = hidx_v[pl.ds(j * LANES, LANES)]
            own = (lax.shift_right_logical(v, 4) & (NS - 1)) == s
            loc = lax.shift_left(lax.shift_right_logical(v, 8), 4) + (v & (RBLK - 1))
            tgt = jnp.where(own, loc, LH_DATA + lane16)
            plsc.addupdate_scatter(hist_v, [tgt], ones16)
            return 0
        lax.fori_loop(0, HCHUNK // LANES, add_j, 0)

    def stream_pass(tbl_hbm, src_hbm, dst_hbm):
        base = c * E + ebase
        idx_end = B * E - CHUNK

        def load_idx(k, ring_row):
            off = jnp.minimum(base + k * CHUNK, idx_end)
            pltpu.sync_copy(src_hbm.at[pl.ds(off, CHUNK)], srcr_v.at[ring_row])
            pltpu.sync_copy(dst_hbm.at[pl.ds(off, CHUNK)], dstr_v.at[ring_row])

            def bias_j(j, _):
                sl = pl.ds(j * LANES, LANES)
                srcr_v[ring_row, sl] = srcr_v[ring_row, sl] + row0
                return 0
            lax.fori_loop(0, CHUNK // LANES, bias_j, 0)

        def gather(k, rows_ref, gsem):
            return pltpu.async_copy(tbl_hbm.at[srcr_v.at[k % 4]], rows_ref, gsem)

        load_idx(0, 0)
        load_idx(1, 1)
        gather(0, rows0_v, gsem0)
        gather(1, rows1_v, gsem1)

        def pair_body(t, _):
            for sub, (rows_ref, gsem, ssem) in enumerate(
                    ((rows0_v, gsem0, ssem0), (rows1_v, gsem1, ssem1))):
                k = 2 * t + sub
                pltpu.make_async_copy(tbl_hbm.at[srcr_v.at[k % 4]], rows_ref, gsem).wait()
                sc = pltpu.async_copy(rows_ref, acc_sh.at[dstr_v.at[k % 4]], ssem, add=True)
                load_idx(k + 2, (k + 2) % 4)
                if sub == 1:
                    @pl.when(t < NH_CHUNKS)
                    def _():
                        hist_chunk(t, dst_hbm)
                sc.wait()
                gather(k + 2, rows_ref, gsem)
            return 0
        lax.fori_loop(0, PAIRS, pair_body, 0)
        pltpu.make_async_copy(tbl_hbm.at[srcr_v.at[NCHUNK % 4]], rows0_v, gsem0).wait()
        pltpu.make_async_copy(tbl_hbm.at[srcr_v.at[(NCHUNK + 1) % 4]], rows1_v, gsem1).wait()

    def normalize(dst_hbm, add_bias):
        def nround(i, _):
            blk = i * NS + s

            @pl.when(blk < NBLK_TOT)
            def _():
                base = blk * RBLK
                pltpu.sync_copy(acc_sh.at[pl.ds(base, RBLK)], nrm_v)

                def nrow(r, _):
                    gi = jnp.full((LANES,), 0, jnp.int32) + (i * RBLK + r)
                    d = plsc.load_gather(hist_v, [gi])
                    recip = jnp.where(d > 0.0, 1.0 / d, 0.0)
                    for j in range(C // LANES):
                        sl = pl.ds(j * LANES, LANES)
                        v = nrm_v[r, sl] * recip
                        if add_bias:
                            v = v + bias_v[sl]
                        nrm_v[r, sl] = v
                    return 0
                lax.fori_loop(0, RBLK, nrow, 0)
                pltpu.sync_copy(nrm_v, dst_hbm.at[pl.ds(row0 + base, RBLK)])
            return 0
        lax.fori_loop(0, BLK_ROUNDS, nround, 0)

    zero_acc()
    zero_hist()
    plsc.subcore_barrier()
    stream_pass(xt_hbm, nidx_hbm, eidx_hbm)
    plsc.subcore_barrier()
    normalize(xe_hbm, add_bias=False)
    zero_acc()
    zero_hist()
    plsc.subcore_barrier()
    stream_pass(xe_hbm, eidx_hbm, nidx_hbm)
    plsc.subcore_barrier()
    normalize(out_hbm, add_bias=True)


def _build_sc_kernel(interpret=False):
    mesh = plsc.VectorSubcoreMesh(
        core_axis_name="c", subcore_axis_name="s", num_cores=NC, num_subcores=NS
    )
    return pl.kernel(
        _sc_body,
        out_type=(
            jax.ShapeDtypeStruct((B * N, C), jnp.float32),
            jax.ShapeDtypeStruct((B * N, C), jnp.float32),
        ),
        mesh=mesh,
        compiler_params=pltpu.CompilerParams(needs_layout_passes=False),
        scratch_types=[
            pltpu.VMEM_SHARED((N, C), jnp.float32),
            pltpu.VMEM((4, CHUNK), jnp.int32),
            pltpu.VMEM((4, CHUNK), jnp.int32),
            pltpu.VMEM((CHUNK, C), jnp.float32),
            pltpu.VMEM((CHUNK, C), jnp.float32),
            pltpu.VMEM((LH,), jnp.float32),
            pltpu.VMEM((HCHUNK,), jnp.int32),
            pltpu.VMEM((RBLK, C), jnp.float32),
            pltpu.VMEM((C,), jnp.float32),
            pltpu.SemaphoreType.DMA,
            pltpu.SemaphoreType.DMA,
            pltpu.SemaphoreType.DMA,
            pltpu.SemaphoreType.DMA,
        ],
        interpret=interpret,
    )


_hyconv_sc = _build_sc_kernel()


def kernel(x, H, theta, bias):
    xt = _project(x, theta).reshape(B * N, C)
    nidx = H[:, 0, :].reshape(-1)
    eidx = H[:, 1, :].reshape(-1)
    out, _ = _hyconv_sc(xt, nidx, eidx, bias)
    return out.reshape(B, N, C)


# speedup vs baseline: 3.0653x; 1.1108x over previous
import functools

import jax
import jax.numpy as jnp
from jax import lax
from jax.experimental import pallas as pl
from jax.experimental.pallas import tpu as pltpu
from jax.experimental.pallas import tpu_sc as plsc

B = 2
N = 10000
E = 320000
C = 128

NC = 2
NS = 16
LANES = 16

E_PER_TILE = E // NS
CHUNK = 80
NCHUNK = E_PER_TILE // CHUNK
PAIRS = NCHUNK // 2

HCHUNK = 4000
NH_CHUNKS = E // HCHUNK

ZBLK = 80
NZBLK = N // ZBLK
Z_ROUNDS = (NZBLK + NS - 1) // NS

RBLK = 16
NBLK_TOT = N // RBLK
BLK_ROUNDS = (NBLK_TOT + NS - 1) // NS
LH_DATA = BLK_ROUNDS * RBLK
LH = LH_DATA + LANES


def _matmul_body(x_ref, th_ref, o_ref):
    o_ref[0] = jnp.dot(x_ref[0], th_ref[...], preferred_element_type=jnp.float32)


def _project(x, theta):
    RB = 1000
    return pl.pallas_call(
        _matmul_body,
        grid=(B, N // RB),
        in_specs=[
            pl.BlockSpec((1, RB, C), lambda b, i: (b, i, 0)),
            pl.BlockSpec((C, C), lambda b, i: (0, 0)),
        ],
        out_specs=pl.BlockSpec((1, RB, C), lambda b, i: (b, i, 0)),
        out_shape=jax.ShapeDtypeStruct((B, N, C), jnp.float32),
    )(x, theta)


def _sc_body(xt_hbm, nidx_hbm, eidx_hbm, bias_hbm, out_hbm, xe_hbm,
             acc_sh, srcr_v, dstr_v, rows0_v, rows1_v, hist_v, hidx_v,
             nrm_v, bias_v, gsem0, gsem1, ssem0, ssem1, hsem):
    c = lax.axis_index("c")
    s = lax.axis_index("s")
    ebase = s * E_PER_TILE
    row0 = c * N

    pltpu.sync_copy(bias_hbm, bias_v)
    ones16 = jnp.ones((LANES,), jnp.float32)
    spill16 = LH_DATA + lax.iota(jnp.int32, LANES)

    def zero_acc():
        def zrow(r, _):
            for j in range(C // LANES):
                rows0_v[r, j * LANES:(j + 1) * LANES] = jnp.zeros((LANES,), jnp.float32)
            return 0
        lax.fori_loop(0, ZBLK, zrow, 0)
        for i in range(Z_ROUNDS):
            blk = i * NS + s

            @pl.when(blk < NZBLK)
            def _():
                pltpu.sync_copy(rows0_v, acc_sh.at[pl.ds(blk * ZBLK, ZBLK)])

    def zero_hist():
        def zh(i, _):
            hist_v[pl.ds(i * LANES, LANES)] = jnp.zeros((LANES,), jnp.float32)
            return 0
        lax.fori_loop(0, LH // LANES, zh, 0)

    def hist_dma(h, dst_hbm, start):
        off = c * E + jnp.minimum(h, NH_CHUNKS - 1) * HCHUNK
        d = pltpu.make_async_copy(dst_hbm.at[pl.ds(off, HCHUNK)],
                                  hidx_v.at[pl.ds((h % 2) * HCHUNK, HCHUNK)], hsem)
        if start:
            d.start()
        return d

    def hist_chunk(h):
        pbase = (h % 2) * HCHUNK

        def add_j(j, _):
            for u in range(5):
                v = hidx_v[pl.ds(pbase + (j * 5 + u) * LANES, LANES)]
                own = (lax.shift_right_logical(v, 4) & (NS - 1)) == s
                loc = lax.shift_left(lax.shift_right_logical(v, 8), 4) + (v & (RBLK - 1))
                tgt = jnp.where(own, loc, spill16)
                plsc.addupdate_scatter(hist_v, [tgt], ones16)
            return 0
        lax.fori_loop(0, HCHUNK // LANES // 5, add_j, 0)

    def stream_pass(tbl_hbm, src_hbm, dst_hbm):
        base = c * E + ebase
        idx_end = B * E - CHUNK

        def load_idx(k, ring_row):
            off = jnp.minimum(base + k * CHUNK, idx_end)
            pltpu.sync_copy(src_hbm.at[pl.ds(off, CHUNK)], srcr_v.at[ring_row])
            pltpu.sync_copy(dst_hbm.at[pl.ds(off, CHUNK)], dstr_v.at[ring_row])

            def bias_j(j, _):
                sl = pl.ds(j * LANES, LANES)
                srcr_v[ring_row, sl] = srcr_v[ring_row, sl] + row0
                return 0
            lax.fori_loop(0, CHUNK // LANES, bias_j, 0)

        def gather(k, rows_ref, gsem):
            return pltpu.async_copy(tbl_hbm.at[srcr_v.at[k % 4]], rows_ref, gsem)

        load_idx(0, 0)
        load_idx(1, 1)
        gather(0, rows0_v, gsem0)
        gather(1, rows1_v, gsem1)
        hist_dma(0, dst_hbm, start=True)

        def pair_body(t, _):
            for sub, (rows_ref, gsem, ssem) in enumerate(
                    ((rows0_v, gsem0, ssem0), (rows1_v, gsem1, ssem1))):
                k = 2 * t + sub
                pltpu.make_async_copy(tbl_hbm.at[srcr_v.at[k % 4]], rows_ref, gsem).wait()
                sc = pltpu.async_copy(rows_ref, acc_sh.at[dstr_v.at[k % 4]], ssem, add=True)
                load_idx(k + 2, (k + 2) % 4)
                if sub == 1:
                    @pl.when(t < NH_CHUNKS)
                    def _():
                        hist_dma(t, dst_hbm, start=False).wait()
                        hist_dma(t + 1, dst_hbm, start=True)
                        hist_chunk(t)
                sc.wait()
                gather(k + 2, rows_ref, gsem)
            return 0
        lax.fori_loop(0, PAIRS, pair_body, 0)
        pltpu.make_async_copy(tbl_hbm.at[srcr_v.at[NCHUNK % 4]], rows0_v, gsem0).wait()
        pltpu.make_async_copy(tbl_hbm.at[srcr_v.at[(NCHUNK + 1) % 4]], rows1_v, gsem1).wait()
        hist_dma(NH_CHUNKS, dst_hbm, start=False).wait()

    def normalize(dst_hbm, add_bias):
        def nround(i, _):
            blk = i * NS + s

            @pl.when(blk < NBLK_TOT)
            def _():
                base = blk * RBLK
                pltpu.sync_copy(acc_sh.at[pl.ds(base, RBLK)], nrm_v)

                def nrow(r, _):
                    gi = jnp.full((LANES,), 0, jnp.int32) + (i * RBLK + r)
                    d = plsc.load_gather(hist_v, [gi])
                    recip = jnp.where(d > 0.0, 1.0 / d, 0.0)
                    for j in range(C // LANES):
                        sl = pl.ds(j * LANES, LANES)
                        v = nrm_v[r, sl] * recip
                        if add_bias:
                            v = v + bias_v[sl]
                        nrm_v[r, sl] = v
                    return 0
                lax.fori_loop(0, RBLK, nrow, 0)
                pltpu.sync_copy(nrm_v, dst_hbm.at[pl.ds(row0 + base, RBLK)])
            return 0
        lax.fori_loop(0, BLK_ROUNDS, nround, 0)

    zero_acc()
    zero_hist()
    plsc.subcore_barrier()
    stream_pass(xt_hbm, nidx_hbm, eidx_hbm)
    plsc.subcore_barrier()
    normalize(xe_hbm, add_bias=False)
    zero_acc()
    zero_hist()
    plsc.subcore_barrier()
    stream_pass(xe_hbm, eidx_hbm, nidx_hbm)
    plsc.subcore_barrier()
    normalize(out_hbm, add_bias=True)


def _build_sc_kernel(interpret=False):
    mesh = plsc.VectorSubcoreMesh(
        core_axis_name="c", subcore_axis_name="s", num_cores=NC, num_subcores=NS
    )
    return pl.kernel(
        _sc_body,
        out_type=(
            jax.ShapeDtypeStruct((B * N, C), jnp.float32),
            jax.ShapeDtypeStruct((B * N, C), jnp.float32),
        ),
        mesh=mesh,
        compiler_params=pltpu.CompilerParams(needs_layout_passes=False),
        scratch_types=[
            pltpu.VMEM_SHARED((N, C), jnp.float32),
            pltpu.VMEM((4, CHUNK), jnp.int32),
            pltpu.VMEM((4, CHUNK), jnp.int32),
            pltpu.VMEM((CHUNK, C), jnp.float32),
            pltpu.VMEM((CHUNK, C), jnp.float32),
            pltpu.VMEM((LH,), jnp.float32),
            pltpu.VMEM((2 * HCHUNK,), jnp.int32),
            pltpu.VMEM((RBLK, C), jnp.float32),
            pltpu.VMEM((C,), jnp.float32),
            pltpu.SemaphoreType.DMA,
            pltpu.SemaphoreType.DMA,
            pltpu.SemaphoreType.DMA,
            pltpu.SemaphoreType.DMA,
            pltpu.SemaphoreType.DMA,
        ],
        interpret=interpret,
    )


_hyconv_sc = _build_sc_kernel()


def kernel(x, H, theta, bias):
    xt = _project(x, theta).reshape(B * N, C)
    nidx = H[:, 0, :].reshape(-1)
    eidx = H[:, 1, :].reshape(-1)
    out, _ = _hyconv_sc(xt, nidx, eidx, bias)
    return out.reshape(B, N, C)


# speedup vs baseline: 3.7631x; 1.2276x over previous
import functools

import jax
import jax.numpy as jnp
from jax import lax
from jax.experimental import pallas as pl
from jax.experimental.pallas import tpu as pltpu
from jax.experimental.pallas import tpu_sc as plsc

B = 2
N = 10000
E = 320000
C = 128

NC = 2
NS = 16
LANES = 16

E_PER_TILE = E // NS
CHUNK = 80
NCHUNK = E_PER_TILE // CHUNK
PAIRS = NCHUNK // 2

HCHUNK = 4000
NH_CHUNKS = E // HCHUNK

ZBLK = 80
NZBLK = N // ZBLK
Z_ROUNDS = (NZBLK + NS - 1) // NS

RBLK = 16
NBLK_TOT = N // RBLK
BLK_ROUNDS = (NBLK_TOT + NS - 1) // NS
LH_DATA = BLK_ROUNDS * RBLK
LH = LH_DATA + LANES


def _matmul_body(x_ref, th_ref, o_ref):
    o_ref[0] = jnp.dot(x_ref[0], th_ref[...], preferred_element_type=jnp.float32)


def _project(x, theta):
    RB = 1000
    return pl.pallas_call(
        _matmul_body,
        grid=(B, N // RB),
        in_specs=[
            pl.BlockSpec((1, RB, C), lambda b, i: (b, i, 0)),
            pl.BlockSpec((C, C), lambda b, i: (0, 0)),
        ],
        out_specs=pl.BlockSpec((1, RB, C), lambda b, i: (b, i, 0)),
        out_shape=jax.ShapeDtypeStruct((B, N, C), jnp.float32),
    )(x, theta)


def _sc_body(xt_hbm, nidx_hbm, eidx_hbm, bias_hbm, out_hbm, xe_hbm,
             acc_sh, srcr_v, dstr_v, rows0_v, rows1_v, hist_v, hidx_v,
             nrm_v, bias_v, gsem0, gsem1, ssem0, ssem1, hsem, isem):
    c = lax.axis_index("c")
    s = lax.axis_index("s")
    ebase = s * E_PER_TILE
    row0 = c * N

    pltpu.sync_copy(bias_hbm, bias_v)
    ones16 = jnp.ones((LANES,), jnp.float32)
    spill16 = LH_DATA + lax.iota(jnp.int32, LANES)

    def zero_acc():
        def zrow(r, _):
            for j in range(C // LANES):
                rows0_v[r, j * LANES:(j + 1) * LANES] = jnp.zeros((LANES,), jnp.float32)
            return 0
        lax.fori_loop(0, ZBLK, zrow, 0)
        for i in range(Z_ROUNDS):
            blk = i * NS + s

            @pl.when(blk < NZBLK)
            def _():
                pltpu.sync_copy(rows0_v, acc_sh.at[pl.ds(blk * ZBLK, ZBLK)])

    def zero_hist():
        def zh(i, _):
            hist_v[pl.ds(i * LANES, LANES)] = jnp.zeros((LANES,), jnp.float32)
            return 0
        lax.fori_loop(0, LH // LANES, zh, 0)

    def hist_dma(h, dst_hbm, start):
        off = c * E + jnp.minimum(h, NH_CHUNKS - 1) * HCHUNK
        d = pltpu.make_async_copy(dst_hbm.at[pl.ds(off, HCHUNK)],
                                  hidx_v.at[pl.ds((h % 2) * HCHUNK, HCHUNK)], hsem)
        if start:
            d.start()
        return d

    def hist_chunk(h):
        pbase = (h % 2) * HCHUNK

        def add_j(j, _):
            for u in range(5):
                v = hidx_v[pl.ds(pbase + (j * 5 + u) * LANES, LANES)]
                own = (lax.shift_right_logical(v, 4) & (NS - 1)) == s
                loc = lax.shift_left(lax.shift_right_logical(v, 8), 4) + (v & (RBLK - 1))
                tgt = jnp.where(own, loc, spill16)
                plsc.addupdate_scatter(hist_v, [tgt], ones16)
            return 0
        lax.fori_loop(0, HCHUNK // LANES // 5, add_j, 0)

    def stream_pass(tbl_hbm, src_hbm, dst_hbm):
        base = c * E + ebase
        idx_end = B * E - CHUNK

        def idx_dmas(k, ring_row):
            off = jnp.minimum(base + k * CHUNK, idx_end)
            return (pltpu.make_async_copy(src_hbm.at[pl.ds(off, CHUNK)],
                                          srcr_v.at[ring_row], isem),
                    pltpu.make_async_copy(dst_hbm.at[pl.ds(off, CHUNK)],
                                          dstr_v.at[ring_row], isem))

        def bias_idx(ring_row):
            def bias_j(j, _):
                sl = pl.ds(j * LANES, LANES)
                srcr_v[ring_row, sl] = srcr_v[ring_row, sl] + row0
                return 0
            lax.fori_loop(0, CHUNK // LANES, bias_j, 0)

        def load_idx(k, ring_row):
            for d in idx_dmas(k, ring_row):
                d.start()
                d.wait()
            bias_idx(ring_row)

        def gather(k, rows_ref, gsem):
            return pltpu.async_copy(tbl_hbm.at[srcr_v.at[k % 4]], rows_ref, gsem)

        load_idx(0, 0)
        load_idx(1, 1)
        gather(0, rows0_v, gsem0)
        gather(1, rows1_v, gsem1)
        hist_dma(0, dst_hbm, start=True)

        def pair_body(t, _):
            for sub, (rows_ref, gsem, ssem) in enumerate(
                    ((rows0_v, gsem0, ssem0), (rows1_v, gsem1, ssem1))):
                k = 2 * t + sub
                pltpu.make_async_copy(tbl_hbm.at[srcr_v.at[k % 4]], rows_ref, gsem).wait()
                sc = pltpu.async_copy(rows_ref, acc_sh.at[dstr_v.at[k % 4]], ssem, add=True)
                da, db = idx_dmas(k + 2, (k + 2) % 4)
                da.start()
                db.start()
                if sub == 1:
                    @pl.when(t < NH_CHUNKS)
                    def _():
                        hist_dma(t, dst_hbm, start=False).wait()
                        hist_dma(t + 1, dst_hbm, start=True)
                        hist_chunk(t)
                sc.wait()
                da.wait()
                db.wait()
                bias_idx((k + 2) % 4)
                gather(k + 2, rows_ref, gsem)
            return 0
        lax.fori_loop(0, PAIRS, pair_body, 0)
        pltpu.make_async_copy(tbl_hbm.at[srcr_v.at[NCHUNK % 4]], rows0_v, gsem0).wait()
        pltpu.make_async_copy(tbl_hbm.at[srcr_v.at[(NCHUNK + 1) % 4]], rows1_v, gsem1).wait()
        hist_dma(NH_CHUNKS, dst_hbm, start=False).wait()

    def normalize(dst_hbm, add_bias):
        def nround(i, _):
            blk = i * NS + s

            @pl.when(blk < NBLK_TOT)
            def _():
                base = blk * RBLK
                pltpu.sync_copy(acc_sh.at[pl.ds(base, RBLK)], nrm_v)

                def nrow(r, _):
                    gi = jnp.full((LANES,), 0, jnp.int32) + (i * RBLK + r)
                    d = plsc.load_gather(hist_v, [gi])
                    recip = jnp.where(d > 0.0, 1.0 / d, 0.0)
                    for j in range(C // LANES):
                        sl = pl.ds(j * LANES, LANES)
                        v = nrm_v[r, sl] * recip
                        if add_bias:
                            v = v + bias_v[sl]
                        nrm_v[r, sl] = v
                    return 0
                lax.fori_loop(0, RBLK, nrow, 0)
                pltpu.sync_copy(nrm_v, dst_hbm.at[pl.ds(row0 + base, RBLK)])
            return 0
        lax.fori_loop(0, BLK_ROUNDS, nround, 0)

    zero_acc()
    zero_hist()
    plsc.subcore_barrier()
    stream_pass(xt_hbm, nidx_hbm, eidx_hbm)
    plsc.subcore_barrier()
    normalize(xe_hbm, add_bias=False)
    zero_acc()
    zero_hist()
    plsc.subcore_barrier()
    stream_pass(xe_hbm, eidx_hbm, nidx_hbm)
    plsc.subcore_barrier()
    normalize(out_hbm, add_bias=True)


def _build_sc_kernel(interpret=False):
    mesh = plsc.VectorSubcoreMesh(
        core_axis_name="c", subcore_axis_name="s", num_cores=NC, num_subcores=NS
    )
    return pl.kernel(
        _sc_body,
        out_type=(
            jax.ShapeDtypeStruct((B * N, C), jnp.float32),
            jax.ShapeDtypeStruct((B * N, C), jnp.float32),
        ),
        mesh=mesh,
        compiler_params=pltpu.CompilerParams(needs_layout_passes=False),
        scratch_types=[
            pltpu.VMEM_SHARED((N, C), jnp.float32),
            pltpu.VMEM((4, CHUNK), jnp.int32),
            pltpu.VMEM((4, CHUNK), jnp.int32),
            pltpu.VMEM((CHUNK, C), jnp.float32),
            pltpu.VMEM((CHUNK, C), jnp.float32),
            pltpu.VMEM((LH,), jnp.float32),
            pltpu.VMEM((2 * HCHUNK,), jnp.int32),
            pltpu.VMEM((RBLK, C), jnp.float32),
            pltpu.VMEM((C,), jnp.float32),
            pltpu.SemaphoreType.DMA,
            pltpu.SemaphoreType.DMA,
            pltpu.SemaphoreType.DMA,
            pltpu.SemaphoreType.DMA,
            pltpu.SemaphoreType.DMA,
            pltpu.SemaphoreType.DMA,
        ],
        interpret=interpret,
    )


_hyconv_sc = _build_sc_kernel()


def kernel(x, H, theta, bias):
    xt = _project(x, theta).reshape(B * N, C)
    nidx = H[:, 0, :].reshape(-1)
    eidx = H[:, 1, :].reshape(-1)
    out, _ = _hyconv_sc(xt, nidx, eidx, bias)
    return out.reshape(B, N, C)


# speedup vs baseline: 3.7645x; 1.0004x over previous
import jax
import jax.numpy as jnp
from jax import lax
from jax.experimental import pallas as pl
from jax.experimental.pallas import tpu as pltpu
from jax.experimental.pallas import tpu_sc as plsc

B = 2
N = 10000
E = 320000
C = 128

NC = 2
NS = 16
LANES = 16

E_PER_TILE = E // NS
CHUNK = 80
NCHUNK = E_PER_TILE // CHUNK
PAIRS = NCHUNK // 2

HCHUNK = 4000
NH_CHUNKS = E // HCHUNK

ZBLK = 80
NZBLK = N // ZBLK
Z_ROUNDS = (NZBLK + NS - 1) // NS

RBLK = 16
NBLK_TOT = N // RBLK
BLK_ROUNDS = (NBLK_TOT + NS - 1) // NS
LH_DATA = BLK_ROUNDS * RBLK
LH = LH_DATA + LANES


def _matmul_body(x_ref, th_ref, o_ref):
    o_ref[0] = jnp.dot(x_ref[0], th_ref[...], preferred_element_type=jnp.float32)


def _project(x, theta):
    RB = 1000
    return pl.pallas_call(
        _matmul_body,
        grid=(B, N // RB),
        in_specs=[
            pl.BlockSpec((1, RB, C), lambda b, i: (b, i, 0)),
            pl.BlockSpec((C, C), lambda b, i: (0, 0)),
        ],
        out_specs=pl.BlockSpec((1, RB, C), lambda b, i: (b, i, 0)),
        out_shape=jax.ShapeDtypeStruct((B, N, C), jnp.float32),
    )(x, theta)


def _sc_body(xt_hbm, nidx_hbm, eidx_hbm, bias_hbm, out_hbm, xe_hbm,
             acc_sh, srcr_v, dstr_v, rows0_v, rows1_v, hist_v, hidx_v,
             nrm_v, bias_v, gsem0, gsem1, ssem0, ssem1, hsem, isem):
    c = lax.axis_index("c")
    s = lax.axis_index("s")
    ebase = s * E_PER_TILE
    row0 = c * N

    pltpu.sync_copy(bias_hbm, bias_v)
    ones16 = jnp.ones((LANES,), jnp.float32)
    spill16 = LH_DATA + lax.iota(jnp.int32, LANES)

    def zero_acc():
        def zrow(r, _):
            for j in range(C // LANES):
                rows0_v[r, j * LANES:(j + 1) * LANES] = jnp.zeros((LANES,), jnp.float32)
            return 0
        lax.fori_loop(0, ZBLK, zrow, 0)
        for i in range(Z_ROUNDS):
            blk = i * NS + s

            @pl.when(blk < NZBLK)
            def _():
                pltpu.sync_copy(rows0_v, acc_sh.at[pl.ds(blk * ZBLK, ZBLK)])

    def zero_hist():
        def zh(i, _):
            hist_v[pl.ds(i * LANES, LANES)] = jnp.zeros((LANES,), jnp.float32)
            return 0
        lax.fori_loop(0, LH // LANES, zh, 0)

    def hist_dma(h, dst_hbm, start):
        off = c * E + jnp.minimum(h, NH_CHUNKS - 1) * HCHUNK
        d = pltpu.make_async_copy(dst_hbm.at[pl.ds(off, HCHUNK)],
                                  hidx_v.at[pl.ds((h % 2) * HCHUNK, HCHUNK)], hsem)
        if start:
            d.start()
        return d

    def hist_chunk(h):
        pbase = (h % 2) * HCHUNK

        def add_j(j, _):
            for u in range(5):
                v = hidx_v[pl.ds(pbase + (j * 5 + u) * LANES, LANES)]
                own = (lax.shift_right_logical(v, 4) & (NS - 1)) == s
                loc = lax.shift_left(lax.shift_right_logical(v, 8), 4) + (v & (RBLK - 1))
                tgt = jnp.where(own, loc, spill16)
                plsc.addupdate_scatter(hist_v, [tgt], ones16)
            return 0
        lax.fori_loop(0, HCHUNK // LANES // 5, add_j, 0)

    def stream_pass(tbl_hbm, src_hbm, dst_hbm):
        base = c * E + ebase
        idx_end = B * E - CHUNK

        def idx_dmas(k, ring_row):
            off = jnp.minimum(base + k * CHUNK, idx_end)
            return (pltpu.make_async_copy(src_hbm.at[pl.ds(off, CHUNK)],
                                          srcr_v.at[ring_row], isem),
                    pltpu.make_async_copy(dst_hbm.at[pl.ds(off, CHUNK)],
                                          dstr_v.at[ring_row], isem))

        def bias_idx(ring_row):
            def bias_j(j, _):
                sl = pl.ds(j * LANES, LANES)
                srcr_v[ring_row, sl] = srcr_v[ring_row, sl] + row0
                return 0
            lax.fori_loop(0, CHUNK // LANES, bias_j, 0)

        def load_idx(k, ring_row):
            for d in idx_dmas(k, ring_row):
                d.start()
                d.wait()
            bias_idx(ring_row)

        def gather(k, rows_ref, gsem):
            return pltpu.async_copy(tbl_hbm.at[srcr_v.at[k % 4]], rows_ref, gsem)

        load_idx(0, 0)
        load_idx(1, 1)
        gather(0, rows0_v, gsem0)
        gather(1, rows1_v, gsem1)
        hist_dma(0, dst_hbm, start=True)

        def pair_body(t, _):
            for sub, (rows_ref, gsem, ssem) in enumerate(
                    ((rows0_v, gsem0, ssem0), (rows1_v, gsem1, ssem1))):
                k = 2 * t + sub
                pltpu.make_async_copy(tbl_hbm.at[srcr_v.at[k % 4]], rows_ref, gsem).wait()
                sc = pltpu.async_copy(rows_ref, acc_sh.at[dstr_v.at[k % 4]], ssem, add=True)
                da, db = idx_dmas(k + 2, (k + 2) % 4)
                da.start()
                db.start()
                if sub == 1:
                    @pl.when(t < NH_CHUNKS)
                    def _():
                        hist_dma(t, dst_hbm, start=False).wait()
                        hist_dma(t + 1, dst_hbm, start=True)
                        hist_chunk(t)
                sc.wait()
                da.wait()
                db.wait()
                bias_idx((k + 2) % 4)
                gather(k + 2, rows_ref, gsem)
            return 0
        lax.fori_loop(0, PAIRS, pair_body, 0)
        pltpu.make_async_copy(tbl_hbm.at[srcr_v.at[NCHUNK % 4]], rows0_v, gsem0).wait()
        pltpu.make_async_copy(tbl_hbm.at[srcr_v.at[(NCHUNK + 1) % 4]], rows1_v, gsem1).wait()
        hist_dma(NH_CHUNKS, dst_hbm, start=False).wait()

    def normalize(dst_hbm, add_bias):
        def nround(i, _):
            blk = i * NS + s

            @pl.when(blk < NBLK_TOT)
            def _():
                base = blk * RBLK
                pltpu.sync_copy(acc_sh.at[pl.ds(base, RBLK)], nrm_v)

                def nrow(r, _):
                    gi = jnp.full((LANES,), 0, jnp.int32) + (i * RBLK + r)
                    d = plsc.load_gather(hist_v, [gi])
                    recip = jnp.where(d > 0.0, 1.0 / d, 0.0)
                    for j in range(C // LANES):
                        sl = pl.ds(j * LANES, LANES)
                        v = nrm_v[r, sl] * recip
                        if add_bias:
                            v = v + bias_v[sl]
                        nrm_v[r, sl] = v
                    return 0
                lax.fori_loop(0, RBLK, nrow, 0)
                pltpu.sync_copy(nrm_v, dst_hbm.at[pl.ds(row0 + base, RBLK)])
            return 0
        lax.fori_loop(0, BLK_ROUNDS, nround, 0)

    zero_acc()
    zero_hist()
    plsc.subcore_barrier()
    stream_pass(xt_hbm, nidx_hbm, eidx_hbm)
    plsc.subcore_barrier()
    normalize(xe_hbm, add_bias=False)
    plsc.subcore_barrier()
    zero_acc()
    zero_hist()
    plsc.subcore_barrier()
    stream_pass(xe_hbm, eidx_hbm, nidx_hbm)
    plsc.subcore_barrier()
    normalize(out_hbm, add_bias=True)


def _build_sc_kernel(interpret=False):
    mesh = plsc.VectorSubcoreMesh(
        core_axis_name="c", subcore_axis_name="s", num_cores=NC, num_subcores=NS
    )
    return pl.kernel(
        _sc_body,
        out_type=(
            jax.ShapeDtypeStruct((B * N, C), jnp.float32),
            jax.ShapeDtypeStruct((B * N, C), jnp.float32),
        ),
        mesh=mesh,
        compiler_params=pltpu.CompilerParams(needs_layout_passes=False),
        scratch_types=[
            pltpu.VMEM_SHARED((N, C), jnp.float32),
            pltpu.VMEM((4, CHUNK), jnp.int32),
            pltpu.VMEM((4, CHUNK), jnp.int32),
            pltpu.VMEM((CHUNK, C), jnp.float32),
            pltpu.VMEM((CHUNK, C), jnp.float32),
            pltpu.VMEM((LH,), jnp.float32),
            pltpu.VMEM((2 * HCHUNK,), jnp.int32),
            pltpu.VMEM((RBLK, C), jnp.float32),
            pltpu.VMEM((C,), jnp.float32),
            pltpu.SemaphoreType.DMA,
            pltpu.SemaphoreType.DMA,
            pltpu.SemaphoreType.DMA,
            pltpu.SemaphoreType.DMA,
            pltpu.SemaphoreType.DMA,
            pltpu.SemaphoreType.DMA,
        ],
        interpret=interpret,
    )


_hyconv_sc = _build_sc_kernel()


def kernel(x, H, theta, bias):
    xt = _project(x, theta).reshape(B * N, C)
    nidx = H[:, 0, :].reshape(-1)
    eidx = H[:, 1, :].reshape(-1)
    out, _ = _hyconv_sc(xt, nidx, eidx, bias)
    return out.reshape(B, N, C)
